# baseline (speedup 1.0000x reference)
import jax
import jax.numpy as jnp
from jax import lax
from jax.experimental import pallas as pl
from jax.experimental.pallas import tpu as pltpu
from jax.experimental.pallas import tpu_sc as plsc

NUM_NODES = 366
NUM_CHANNELS = 16
STRIDE = 33
TAB_WORDS = 12080

NC, NS, L = 2, 16, 16
NW = NC * NS
B = 16384 * 50
ROWS_PER_W = B // NW
CHUNK = 2560
NCHUNK = ROWS_PER_W // CHUNK
GROUPS = CHUNK // L


def _sc_body(y_hbm, tab_hbm, out_hbm, tab_v,
             y_v0, y_v1, out_v0, out_v1,
             in_sem0, in_sem1, out_sem0, out_sem1):
    wid = lax.axis_index("c") * NS + lax.axis_index("s")
    row0 = wid * ROWS_PER_W
    y_bufs = (y_v0, y_v1)
    out_bufs = (out_v0, out_v1)
    in_sems = (in_sem0, in_sem1)
    out_sems = (out_sem0, out_sem1)
    pltpu.sync_copy(tab_hbm, tab_v)

    def in_copy(k, b):
        return pltpu.make_async_copy(
            y_hbm.at[pl.ds(row0 + k * CHUNK, CHUNK)], y_bufs[b], in_sems[b])

    def out_copy(k, b):
        return pltpu.make_async_copy(
            out_bufs[b], out_hbm.at[:, pl.ds(row0 + k * CHUNK, CHUNK)],
            out_sems[b])

    def make_group_body(y_v, out_v):
        def one_group(g):
            scaled = y_v[pl.ds(g * L, L)]
            li = scaled.astype(jnp.int32)
            alpha = scaled - li.astype(jnp.float32)
            i1 = li * STRIDE
            r0 = g * L
            ga = [plsc.load_gather(tab_v, [i1 + c])
                  for c in range(NUM_CHANNELS)]
            gb = [plsc.load_gather(tab_v, [i1 + (NUM_CHANNELS + c)])
                  for c in range(NUM_CHANNELS)]
            for c in range(NUM_CHANNELS):
                out_v[c, pl.ds(r0, L)] = ga[c] + alpha * gb[c]

        def group_body(h, _):
            one_group(h * 2)
            one_group(h * 2 + 1)
            return 0
        return group_body

    in_copy(0, 0).start()
    for k in range(NCHUNK):
        b = k % 2
        if k + 1 < NCHUNK:
            in_copy(k + 1, 1 - b).start()
        in_copy(k, b).wait()
        if k >= 2:
            out_copy(k - 2, b).wait()
        lax.fori_loop(0, GROUPS // 2,
                      make_group_body(y_bufs[b], out_bufs[b]), 0)
        out_copy(k, b).start()
    out_copy(NCHUNK - 2, NCHUNK % 2).wait()
    out_copy(NCHUNK - 1, (NCHUNK - 1) % 2).wait()


@jax.jit
def _run(y_flat, comb_flat):
    mesh = plsc.VectorSubcoreMesh(core_axis_name="c", subcore_axis_name="s")
    f = pl.kernel(
        _sc_body,
        out_type=jax.ShapeDtypeStruct((NUM_CHANNELS, B), jnp.float32),
        mesh=mesh,
        scratch_types=[
            pltpu.VMEM((TAB_WORDS,), jnp.float32),
            pltpu.VMEM((CHUNK,), jnp.float32),
            pltpu.VMEM((CHUNK,), jnp.float32),
            pltpu.VMEM((NUM_CHANNELS, CHUNK), jnp.float32),
            pltpu.VMEM((NUM_CHANNELS, CHUNK), jnp.float32),
            pltpu.SemaphoreType.DMA,
            pltpu.SemaphoreType.DMA,
            pltpu.SemaphoreType.DMA,
            pltpu.SemaphoreType.DMA,
        ],
        compiler_params=pltpu.CompilerParams(needs_layout_passes=False),
    )
    return f(y_flat, comb_flat)


def kernel(yday, embedding):
    upper = jnp.roll(embedding, -1, axis=0)
    comb = jnp.concatenate(
        [upper, embedding - upper,
         jnp.zeros((NUM_NODES, STRIDE - 2 * NUM_CHANNELS), jnp.float32)],
        axis=1)
    comb_flat = jnp.pad(comb.reshape(-1), (0, TAB_WORDS - NUM_NODES * STRIDE))
    scaled = yday.reshape(-1) * jnp.float32(NUM_NODES)
    out_t = _run(scaled, comb_flat)
    return out_t.T


# speedup vs baseline: 1.4029x; 1.4029x over previous
import jax
import jax.numpy as jnp
from jax import lax
from jax.experimental import pallas as pl
from jax.experimental.pallas import tpu as pltpu
from jax.experimental.pallas import tpu_sc as plsc

NUM_NODES = 366
NUM_CHANNELS = 16
NPAIR = NUM_CHANNELS // 2
STRIDE = 17
TAB_WORDS = 6224

NC, NS, L = 2, 16, 16
NW = NC * NS
YROWS, YCOLS = 16384, 50
B = YROWS * YCOLS
ROWS_PER_W = B // NW
YR_PER_W = YROWS // NW
PIECE = 128
NPIECE = YR_PER_W // PIECE
CHUNK = 3200
CPP = (PIECE * YCOLS) // CHUNK
NCHUNK = NPIECE * CPP
YR_PER_CHUNK = CHUNK // YCOLS

_HI = -65536


def _sc_body(yt_hbm, tab_hbm, out_hbm, tab_v,
             y_v, out_v0, out_v1,
             out_sem0, out_sem1):
    wid = lax.axis_index("c") * NS + lax.axis_index("s")
    lane0 = wid * YR_PER_W
    row0 = wid * ROWS_PER_W
    out_bufs = (out_v0, out_v1)
    out_sems = (out_sem0, out_sem1)
    pltpu.sync_copy(tab_hbm, tab_v)

    def in_copy(p):
        pltpu.sync_copy(yt_hbm.at[:, pl.ds(lane0 + p * PIECE, PIECE)], y_v)

    def out_copy(k, b):
        return pltpu.make_async_copy(
            out_bufs[b], out_hbm.at[:, pl.ds(row0 + k * CHUNK, CHUNK)],
            out_sems[b])

    def bf16_pair(word):
        lo = plsc.bitcast(lax.shift_left(word, 16), jnp.float32)
        hi = plsc.bitcast(lax.bitwise_and(word, jnp.int32(_HI)), jnp.float32)
        return lo, hi

    lane_stride = lax.broadcasted_iota(jnp.int32, (L,), 0) * YCOLS
    ch_idx = [jnp.full((L,), ch, jnp.int32) for ch in range(NUM_CHANNELS)]

    def compute_chunk(out_v, c):
        @plsc.parallel_loop(0, (YR_PER_CHUNK // L) * YCOLS, step=1, unroll=2)
        def _(g):
            col = lax.shift_right_logical(g, 2)
            h = lax.bitwise_and(g, 3)
            scaled = y_v[col, pl.ds(c * YR_PER_CHUNK + h * L, L)] \
                * jnp.float32(NUM_NODES)
            li = scaled.astype(jnp.int32)
            alpha = scaled - li.astype(jnp.float32)
            i1 = li * STRIDE
            sidx = (h * (L * YCOLS) + col) + lane_stride
            for w in range(NPAIR):
                a_lo, a_hi = bf16_pair(plsc.load_gather(tab_v, [i1 + w]))
                d_lo, d_hi = bf16_pair(
                    plsc.load_gather(tab_v, [i1 + (NPAIR + w)]))
                plsc.store_scatter(out_v, [ch_idx[2 * w], sidx],
                                   a_lo + alpha * d_lo)
                plsc.store_scatter(out_v, [ch_idx[2 * w + 1], sidx],
                                   a_hi + alpha * d_hi)

    for p in range(NPIECE):
        in_copy(p)
        for c in range(CPP):
            k = p * CPP + c
            bo = k % 2
            if k >= 2:
                out_copy(k - 2, bo).wait()
            compute_chunk(out_bufs[bo], c)
            out_copy(k, bo).start()
    out_copy(NCHUNK - 2, NCHUNK % 2).wait()
    out_copy(NCHUNK - 1, (NCHUNK - 1) % 2).wait()


@jax.jit
def _run(yt, tab_packed):
    mesh = plsc.VectorSubcoreMesh(core_axis_name="c", subcore_axis_name="s")
    f = pl.kernel(
        _sc_body,
        out_type=jax.ShapeDtypeStruct((NUM_CHANNELS, B), jnp.float32),
        mesh=mesh,
        scratch_types=[
            pltpu.VMEM((TAB_WORDS,), jnp.int32),
            pltpu.VMEM((YCOLS, PIECE), jnp.float32),
            pltpu.VMEM((NUM_CHANNELS, CHUNK), jnp.float32),
            pltpu.VMEM((NUM_CHANNELS, CHUNK), jnp.float32),
            pltpu.SemaphoreType.DMA,
            pltpu.SemaphoreType.DMA,
        ],
        compiler_params=pltpu.CompilerParams(needs_layout_passes=False),
    )
    return f(yt, tab_packed)


def _pack_pairs(x16):
    bits = lax.bitcast_convert_type(x16.astype(jnp.bfloat16), jnp.uint16)
    bits = bits.astype(jnp.uint32)
    return (bits[:, 0::2] | (bits[:, 1::2] << 16)).astype(jnp.int32)


def kernel(yday, embedding):
    upper = jnp.roll(embedding, -1, axis=0)
    tab = jnp.concatenate(
        [_pack_pairs(upper), _pack_pairs(embedding - upper),
         jnp.zeros((NUM_NODES, STRIDE - 2 * NPAIR), jnp.int32)],
        axis=1)
    tab_flat = jnp.pad(tab.reshape(-1),
                       (0, TAB_WORDS - NUM_NODES * STRIDE))
    out_t = _run(yday.T, tab_flat)
    return out_t.T


# speedup vs baseline: 1.4391x; 1.0258x over previous
import jax
import jax.numpy as jnp
from jax import lax
from jax.experimental import pallas as pl
from jax.experimental.pallas import tpu as pltpu
from jax.experimental.pallas import tpu_sc as plsc

NUM_NODES = 366
NUM_CHANNELS = 16
NPAIR = NUM_CHANNELS // 2
STRIDE = 17
TAB_WORDS = 6224

NC, NS, L = 2, 16, 16
NW = NC * NS
YROWS, YCOLS = 16384, 50
B = YROWS * YCOLS
ROWS_PER_W = B // NW
YR_PER_W = YROWS // NW
PIECE = 128
NPIECE = YR_PER_W // PIECE
CHUNK = 3200
CPP = (PIECE * YCOLS) // CHUNK
NCHUNK = NPIECE * CPP
YR_PER_CHUNK = CHUNK // YCOLS

_HI = -65536


def _sc_body(yt_hbm, tab_hbm, out_hbm, tab_v,
             y_v, out_v0, out_v1,
             out_sem0, out_sem1):
    wid = lax.axis_index("c") * NS + lax.axis_index("s")
    lane0 = wid * YR_PER_W
    row0 = wid * ROWS_PER_W
    out_bufs = (out_v0, out_v1)
    out_sems = (out_sem0, out_sem1)
    pltpu.sync_copy(tab_hbm, tab_v)

    def in_copy(p):
        pltpu.sync_copy(yt_hbm.at[:, pl.ds(lane0 + p * PIECE, PIECE)], y_v)

    def out_copy(k, b):
        return pltpu.make_async_copy(
            out_bufs[b], out_hbm.at[:, pl.ds(row0 + k * CHUNK, CHUNK)],
            out_sems[b])

    def bf16_pair(word):
        lo = plsc.bitcast(lax.shift_left(word, 16), jnp.float32)
        hi = plsc.bitcast(lax.bitwise_and(word, jnp.int32(_HI)), jnp.float32)
        return lo, hi

    lane_stride = lax.broadcasted_iota(jnp.int32, (L,), 0) * YCOLS
    ch_idx = [jnp.full((L,), ch, jnp.int32) for ch in range(NUM_CHANNELS)]

    def compute_chunk(out_v, c):
        @plsc.parallel_loop(0, (YR_PER_CHUNK // L) * YCOLS, step=1, unroll=3)
        def _(g):
            col = lax.shift_right_logical(g, 2)
            h = lax.bitwise_and(g, 3)
            scaled = y_v[col, pl.ds(c * YR_PER_CHUNK + h * L, L)] \
                * jnp.float32(NUM_NODES)
            li = scaled.astype(jnp.int32)
            alpha = scaled - li.astype(jnp.float32)
            i1 = li * STRIDE
            sidx = (h * (L * YCOLS) + col) + lane_stride
            for w in range(NPAIR):
                a_lo, a_hi = bf16_pair(plsc.load_gather(tab_v, [i1 + w]))
                d_lo, d_hi = bf16_pair(
                    plsc.load_gather(tab_v, [i1 + (NPAIR + w)]))
                plsc.store_scatter(out_v, [ch_idx[2 * w], sidx],
                                   a_lo + alpha * d_lo)
                plsc.store_scatter(out_v, [ch_idx[2 * w + 1], sidx],
                                   a_hi + alpha * d_hi)

    for p in range(NPIECE):
        in_copy(p)
        for c in range(CPP):
            k = p * CPP + c
            bo = k % 2
            if k >= 2:
                out_copy(k - 2, bo).wait()
            compute_chunk(out_bufs[bo], c)
            out_copy(k, bo).start()
    out_copy(NCHUNK - 2, NCHUNK % 2).wait()
    out_copy(NCHUNK - 1, (NCHUNK - 1) % 2).wait()


@jax.jit
def _run(yt, tab_packed):
    mesh = plsc.VectorSubcoreMesh(core_axis_name="c", subcore_axis_name="s")
    f = pl.kernel(
        _sc_body,
        out_type=jax.ShapeDtypeStruct((NUM_CHANNELS, B), jnp.float32),
        mesh=mesh,
        scratch_types=[
            pltpu.VMEM((TAB_WORDS,), jnp.int32),
            pltpu.VMEM((YCOLS, PIECE), jnp.float32),
            pltpu.VMEM((NUM_CHANNELS, CHUNK), jnp.float32),
            pltpu.VMEM((NUM_CHANNELS, CHUNK), jnp.float32),
            pltpu.SemaphoreType.DMA,
            pltpu.SemaphoreType.DMA,
        ],
        compiler_params=pltpu.CompilerParams(needs_layout_passes=False),
    )
    return f(yt, tab_packed)


def _pack_pairs(x16):
    bits = lax.bitcast_convert_type(x16.astype(jnp.bfloat16), jnp.uint16)
    bits = bits.astype(jnp.uint32)
    return (bits[:, 0::2] | (bits[:, 1::2] << 16)).astype(jnp.int32)


def kernel(yday, embedding):
    upper = jnp.roll(embedding, -1, axis=0)
    tab = jnp.concatenate(
        [_pack_pairs(upper), _pack_pairs(embedding - upper),
         jnp.zeros((NUM_NODES, STRIDE - 2 * NPAIR), jnp.int32)],
        axis=1)
    tab_flat = jnp.pad(tab.reshape(-1),
                       (0, TAB_WORDS - NUM_NODES * STRIDE))
    out_t = _run(yday.T, tab_flat)
    return out_t.T
